# baseline (speedup 1.0000x reference)
import functools

import jax
import jax.numpy as jnp
from jax import lax
from jax.experimental import pallas as pl
from jax.experimental.pallas import tpu as pltpu
from jax.experimental.pallas import tpu_sc as plsc

N = 10000
D = 128
E = 320000
NW = 32
CH = 128
KM = 128
K0 = 116
K1 = 42
EPAD = 16 * (K0 + K1) * CH
A = 10112
ZR = A // 16

_mesh = plsc.VectorSubcoreMesh(core_axis_name="c", subcore_axis_name="s")


@functools.partial(
    pl.kernel,
    out_type=jax.ShapeDtypeStruct((2, A, D), jnp.float32),
    mesh=_mesh,
    scratch_types=[
        pltpu.VMEM((KM // 2, CH), jnp.int32),
        pltpu.VMEM((KM // 2, CH), jnp.int32),
        pltpu.VMEM((CH,), jnp.int32),
        pltpu.VMEM((CH,), jnp.int32),
        pltpu.VMEM((CH,), jnp.int32),
        pltpu.VMEM((CH,), jnp.int32),
        pltpu.VMEM((CH, D), jnp.float32),
        pltpu.VMEM((CH, D), jnp.float32),
        pltpu.VMEM_SHARED((A, D), jnp.float32),
        pltpu.SemaphoreType.DMA,
        pltpu.SemaphoreType.DMA,
        pltpu.SemaphoreType.DMA,
        pltpu.SemaphoreType.DMA,
    ],
)
def _sc_segsum(p_hbm, src_hbm, dst_hbm, out_hbm,
               src_v, dst_v, sstag0_v, sstag1_v, dstag0_v, dstag1_v,
               rows0_v, rows1_v, acc_sh, semg0, semg1, sems0, sems1):
    c = lax.axis_index("c")
    s = lax.axis_index("s")
    wid = s * 2 + c
    kc2 = jnp.where(c == 0, K0 // 2, K1 // 2)
    z16 = jnp.zeros((16,), jnp.float32)

    def zrow(t, carry):
        for l in range(8):
            rows0_v[t, pl.ds(l * 16, 16)] = z16
        return carry

    lax.fori_loop(0, CH, zrow, 0)
    for k in range(4):
        pltpu.sync_copy(rows0_v, acc_sh.at[pl.ds(s * ZR + k * CH, CH)])
    pltpu.sync_copy(rows0_v.at[pl.ds(0, ZR - 4 * CH), :],
                    acc_sh.at[pl.ds(s * ZR + 4 * CH, ZR - 4 * CH)])
    pltpu.sync_copy(src_hbm.at[pl.ds(wid * (KM // 2), KM // 2), :], src_v)
    pltpu.sync_copy(dst_hbm.at[pl.ds(wid * (KM // 2), KM // 2), :], dst_v)
    plsc.subcore_barrier()

    def unpack(ref, i, col0, stag):
        for t in range(4):
            w = ref[i, pl.ds(col0 + t * 16, 16)]
            stag[pl.ds(t * 16, 16)] = lax.bitwise_and(w, 0xFFFF)
            stag[pl.ds(64 + t * 16, 16)] = lax.shift_right_logical(w, 16)

    unpack(src_v, 0, 0, sstag0_v)
    pltpu.async_copy(p_hbm.at[sstag0_v], rows0_v, semg0)
    unpack(src_v, 0, 64, sstag1_v)
    pltpu.async_copy(p_hbm.at[sstag1_v], rows1_v, semg1)

    def body(i, carry):
        ii = lax.rem(i + 1, kc2)
        pltpu.make_async_copy(p_hbm.at[sstag0_v], rows0_v, semg0).wait()
        unpack(dst_v, i, 0, dstag0_v)
        pltpu.async_copy(rows0_v, acc_sh.at[dstag0_v], sems0, add=True)
        unpack(src_v, ii, 0, sstag0_v)
        pltpu.make_async_copy(rows0_v, acc_sh.at[dstag0_v], sems0).wait()
        pltpu.async_copy(p_hbm.at[sstag0_v], rows0_v, semg0)
        pltpu.make_async_copy(p_hbm.at[sstag1_v], rows1_v, semg1).wait()
        unpack(dst_v, i, 64, dstag1_v)
        pltpu.async_copy(rows1_v, acc_sh.at[dstag1_v], sems1, add=True)
        unpack(src_v, ii, 64, sstag1_v)
        pltpu.make_async_copy(rows1_v, acc_sh.at[dstag1_v], sems1).wait()
        pltpu.async_copy(p_hbm.at[sstag1_v], rows1_v, semg1)
        return carry

    lax.fori_loop(0, kc2, body, 0)
    pltpu.make_async_copy(p_hbm.at[sstag0_v], rows0_v, semg0).wait()
    pltpu.make_async_copy(p_hbm.at[sstag1_v], rows1_v, semg1).wait()
    plsc.subcore_barrier()
    pltpu.sync_copy(acc_sh.at[pl.ds(s * ZR, ZR)],
                    out_hbm.at[c, pl.ds(s * ZR, ZR)])


_R = 400


def _tc_stage1_body(h_ref, wr_ref, wo_ref, b_ref, p_ref, root_ref):
    h = h_ref[...]
    p_ref[...] = jnp.dot(h, wr_ref[...], preferred_element_type=jnp.float32)
    root_ref[...] = (jnp.dot(h, wo_ref[...], preferred_element_type=jnp.float32)
                     + b_ref[...])


def _tc_mid_body(a0_ref, a1_ref, r_ref, wr_ref, wo_ref, b_ref, p_ref, root_ref):
    h = jnp.maximum(a0_ref[...] + a1_ref[...] + r_ref[...], 0.0)
    p_ref[...] = jnp.dot(h, wr_ref[...], preferred_element_type=jnp.float32)
    root_ref[...] = (jnp.dot(h, wo_ref[...], preferred_element_type=jnp.float32)
                     + b_ref[...])


def _tc_final_body(a0_ref, a1_ref, r_ref, ax_ref, o_ref):
    o_ref[...] = a0_ref[...] + a1_ref[...] + r_ref[...] + ax_ref[...]


def _rows_spec(din):
    return pl.BlockSpec((_R, din), lambda i: (i, 0))


def _full_spec(din):
    return pl.BlockSpec((din, D), lambda i: (0, 0))


_B_SPEC = pl.BlockSpec((1, D), lambda i: (0, 0))


def _tc_stage1(h, wr, wo, b):
    return pl.pallas_call(
        _tc_stage1_body,
        grid=(N // _R,),
        in_specs=[_rows_spec(h.shape[1]), _full_spec(h.shape[1]),
                  _full_spec(h.shape[1]), _B_SPEC],
        out_specs=[_rows_spec(D), _rows_spec(D)],
        out_shape=[jax.ShapeDtypeStruct((N, D), jnp.float32),
                   jax.ShapeDtypeStruct((N, D), jnp.float32)],
    )(h, wr, wo, b)


def _tc_mid(a0, a1, r, wr, wo, b):
    return pl.pallas_call(
        _tc_mid_body,
        grid=(N // _R,),
        in_specs=[_rows_spec(D), _rows_spec(D), _rows_spec(D),
                  _full_spec(D), _full_spec(D), _B_SPEC],
        out_specs=[_rows_spec(D), _rows_spec(D)],
        out_shape=[jax.ShapeDtypeStruct((N, D), jnp.float32),
                   jax.ShapeDtypeStruct((N, D), jnp.float32)],
    )(a0, a1, r, wr, wo, b)


def _tc_final(a0, a1, r, ax):
    return pl.pallas_call(
        _tc_final_body,
        grid=(N // _R,),
        in_specs=[_rows_spec(D), _rows_spec(D), _rows_spec(D), _rows_spec(D)],
        out_specs=_rows_spec(D),
        out_shape=jax.ShapeDtypeStruct((N, D), jnp.float32),
    )(a0, a1, r, ax)


def kernel(x, edge_index, additional_x, W_rel1, W_root1, b1,
           W_rel2, W_root2, b2, W_rel3, W_root3, b3):
    h0 = jnp.concatenate([x, additional_x], axis=1)
    src = edge_index[0]
    dst = edge_index[1]
    pad = EPAD - E

    def edge_layout(idx, fill):
        flat = jnp.concatenate([idx, jnp.full((pad,), fill, jnp.int32)])
        cut = 16 * K0 * CH
        e0 = flat[:cut].reshape(16, K0, CH)
        e1 = flat[cut:].reshape(16, K1, CH)
        e0 = jnp.pad(e0, ((0, 0), (0, KM - K0), (0, 0)), constant_values=fill)
        e1 = jnp.pad(e1, ((0, 0), (0, KM - K1), (0, 0)), constant_values=fill)
        e = jnp.stack([e0, e1], axis=1).reshape(NW, KM, CH)
        return (e[:, :, :64] | (e[:, :, 64:] << 16)).reshape(NW * (KM // 2), CH)

    src_p = edge_layout(src, 0)
    dst_p = edge_layout(dst, N)

    p1, root1 = _tc_stage1(h0, W_rel1, W_root1, b1.reshape(1, D))
    acc = _sc_segsum(p1, src_p, dst_p)
    p2, root2 = _tc_mid(acc[0, :N], acc[1, :N], root1, W_rel2, W_root2,
                        b2.reshape(1, D))
    acc = _sc_segsum(p2, src_p, dst_p)
    p3, root3 = _tc_mid(acc[0, :N], acc[1, :N], root2, W_rel3, W_root3,
                        b3.reshape(1, D))
    acc = _sc_segsum(p3, src_p, dst_p)
    return _tc_final(acc[0, :N], acc[1, :N], root3, additional_x)


# speedup vs baseline: 1.0061x; 1.0061x over previous
import functools

import jax
import jax.numpy as jnp
from jax import lax
from jax.experimental import pallas as pl
from jax.experimental.pallas import tpu as pltpu
from jax.experimental.pallas import tpu_sc as plsc

N = 10000
D = 128
E = 320000
NW = 32
CH = 128
KM = 128
K0 = 124
K1 = 34
EPAD = 16 * (K0 + K1) * CH
A = 10112
ZR = A // 16

_mesh = plsc.VectorSubcoreMesh(core_axis_name="c", subcore_axis_name="s")


@functools.partial(
    pl.kernel,
    out_type=jax.ShapeDtypeStruct((2, A, D), jnp.float32),
    mesh=_mesh,
    scratch_types=[
        pltpu.VMEM((KM // 2, CH), jnp.int32),
        pltpu.VMEM((KM // 2, CH), jnp.int32),
        pltpu.VMEM((CH,), jnp.int32),
        pltpu.VMEM((CH,), jnp.int32),
        pltpu.VMEM((CH,), jnp.int32),
        pltpu.VMEM((CH,), jnp.int32),
        pltpu.VMEM((CH, D), jnp.float32),
        pltpu.VMEM((CH, D), jnp.float32),
        pltpu.VMEM_SHARED((A, D), jnp.float32),
        pltpu.SemaphoreType.DMA,
        pltpu.SemaphoreType.DMA,
        pltpu.SemaphoreType.DMA,
        pltpu.SemaphoreType.DMA,
    ],
)
def _sc_segsum(p_hbm, src_hbm, dst_hbm, zeros_hbm, out_hbm,
               src_v, dst_v, sstag0_v, sstag1_v, dstag0_v, dstag1_v,
               rows0_v, rows1_v, acc_sh, semg0, semg1, sems0, sems1):
    c = lax.axis_index("c")
    s = lax.axis_index("s")
    wid = s * 2 + c
    kc2 = jnp.where(c == 0, K0 // 2, K1 // 2)
    pltpu.sync_copy(zeros_hbm, acc_sh.at[pl.ds(s * ZR, ZR)])
    pltpu.sync_copy(src_hbm.at[pl.ds(wid * (KM // 2), KM // 2), :], src_v)
    pltpu.sync_copy(dst_hbm.at[pl.ds(wid * (KM // 2), KM // 2), :], dst_v)
    plsc.subcore_barrier()

    def unpack(ref, i, col0, stag):
        for t in range(4):
            w = ref[i, pl.ds(col0 + t * 16, 16)]
            stag[pl.ds(t * 16, 16)] = lax.bitwise_and(w, 0xFFFF)
            stag[pl.ds(64 + t * 16, 16)] = lax.shift_right_logical(w, 16)

    unpack(src_v, 0, 0, sstag0_v)
    pltpu.async_copy(p_hbm.at[sstag0_v], rows0_v, semg0)
    unpack(src_v, 0, 64, sstag1_v)
    pltpu.async_copy(p_hbm.at[sstag1_v], rows1_v, semg1)

    def body(i, carry):
        ii = lax.rem(i + 1, kc2)
        pltpu.make_async_copy(p_hbm.at[sstag0_v], rows0_v, semg0).wait()
        unpack(dst_v, i, 0, dstag0_v)
        pltpu.async_copy(rows0_v, acc_sh.at[dstag0_v], sems0, add=True)
        unpack(src_v, ii, 0, sstag0_v)
        pltpu.make_async_copy(rows0_v, acc_sh.at[dstag0_v], sems0).wait()
        pltpu.async_copy(p_hbm.at[sstag0_v], rows0_v, semg0)
        pltpu.make_async_copy(p_hbm.at[sstag1_v], rows1_v, semg1).wait()
        unpack(dst_v, i, 64, dstag1_v)
        pltpu.async_copy(rows1_v, acc_sh.at[dstag1_v], sems1, add=True)
        unpack(src_v, ii, 64, sstag1_v)
        pltpu.make_async_copy(rows1_v, acc_sh.at[dstag1_v], sems1).wait()
        pltpu.async_copy(p_hbm.at[sstag1_v], rows1_v, semg1)
        return carry

    lax.fori_loop(0, kc2, body, 0)
    pltpu.make_async_copy(p_hbm.at[sstag0_v], rows0_v, semg0).wait()
    pltpu.make_async_copy(p_hbm.at[sstag1_v], rows1_v, semg1).wait()
    plsc.subcore_barrier()
    pltpu.sync_copy(acc_sh.at[pl.ds(s * ZR, ZR)],
                    out_hbm.at[c, pl.ds(s * ZR, ZR)])


_R = 400


def _tc_stage1_body(h_ref, wr_ref, wo_ref, b_ref, p_ref, root_ref):
    h = h_ref[...]
    p_ref[...] = jnp.dot(h, wr_ref[...], preferred_element_type=jnp.float32)
    root_ref[...] = (jnp.dot(h, wo_ref[...], preferred_element_type=jnp.float32)
                     + b_ref[...])


def _tc_mid_body(a0_ref, a1_ref, r_ref, wr_ref, wo_ref, b_ref, p_ref, root_ref):
    h = jnp.maximum(a0_ref[...] + a1_ref[...] + r_ref[...], 0.0)
    p_ref[...] = jnp.dot(h, wr_ref[...], preferred_element_type=jnp.float32)
    root_ref[...] = (jnp.dot(h, wo_ref[...], preferred_element_type=jnp.float32)
                     + b_ref[...])


def _tc_final_body(a0_ref, a1_ref, r_ref, ax_ref, o_ref):
    o_ref[...] = a0_ref[...] + a1_ref[...] + r_ref[...] + ax_ref[...]


def _rows_spec(din):
    return pl.BlockSpec((_R, din), lambda i: (i, 0))


def _full_spec(din):
    return pl.BlockSpec((din, D), lambda i: (0, 0))


_B_SPEC = pl.BlockSpec((1, D), lambda i: (0, 0))


def _tc_stage1(h, wr, wo, b):
    return pl.pallas_call(
        _tc_stage1_body,
        grid=(N // _R,),
        in_specs=[_rows_spec(h.shape[1]), _full_spec(h.shape[1]),
                  _full_spec(h.shape[1]), _B_SPEC],
        out_specs=[_rows_spec(D), _rows_spec(D)],
        out_shape=[jax.ShapeDtypeStruct((N, D), jnp.float32),
                   jax.ShapeDtypeStruct((N, D), jnp.float32)],
    )(h, wr, wo, b)


def _tc_mid(a0, a1, r, wr, wo, b):
    return pl.pallas_call(
        _tc_mid_body,
        grid=(N // _R,),
        in_specs=[_rows_spec(D), _rows_spec(D), _rows_spec(D),
                  _full_spec(D), _full_spec(D), _B_SPEC],
        out_specs=[_rows_spec(D), _rows_spec(D)],
        out_shape=[jax.ShapeDtypeStruct((N, D), jnp.float32),
                   jax.ShapeDtypeStruct((N, D), jnp.float32)],
    )(a0, a1, r, wr, wo, b)


def _tc_final(a0, a1, r, ax):
    return pl.pallas_call(
        _tc_final_body,
        grid=(N // _R,),
        in_specs=[_rows_spec(D), _rows_spec(D), _rows_spec(D), _rows_spec(D)],
        out_specs=_rows_spec(D),
        out_shape=jax.ShapeDtypeStruct((N, D), jnp.float32),
    )(a0, a1, r, ax)


def kernel(x, edge_index, additional_x, W_rel1, W_root1, b1,
           W_rel2, W_root2, b2, W_rel3, W_root3, b3):
    h0 = jnp.concatenate([x, additional_x], axis=1)
    src = edge_index[0]
    dst = edge_index[1]
    pad = EPAD - E

    def edge_layout(idx, fill):
        flat = jnp.concatenate([idx, jnp.full((pad,), fill, jnp.int32)])
        cut = 16 * K0 * CH
        e0 = flat[:cut].reshape(16, K0, CH)
        e1 = flat[cut:].reshape(16, K1, CH)
        e0 = jnp.pad(e0, ((0, 0), (0, KM - K0), (0, 0)), constant_values=fill)
        e1 = jnp.pad(e1, ((0, 0), (0, KM - K1), (0, 0)), constant_values=fill)
        e = jnp.stack([e0, e1], axis=1).reshape(NW, KM, CH)
        return (e[:, :, :64] | (e[:, :, 64:] << 16)).reshape(NW * (KM // 2), CH)

    src_p = edge_layout(src, 0)
    dst_p = edge_layout(dst, N)
    zeros = jnp.zeros((ZR, D), jnp.float32)

    p1, root1 = _tc_stage1(h0, W_rel1, W_root1, b1.reshape(1, D))
    acc = _sc_segsum(p1, src_p, dst_p, zeros)
    p2, root2 = _tc_mid(acc[0, :N], acc[1, :N], root1, W_rel2, W_root2,
                        b2.reshape(1, D))
    acc = _sc_segsum(p2, src_p, dst_p, zeros)
    p3, root3 = _tc_mid(acc[0, :N], acc[1, :N], root2, W_rel3, W_root3,
                        b3.reshape(1, D))
    acc = _sc_segsum(p3, src_p, dst_p, zeros)
    return _tc_final(acc[0, :N], acc[1, :N], root3, additional_x)


# speedup vs baseline: 1.0644x; 1.0579x over previous
import functools

import jax
import jax.numpy as jnp
from jax import lax
from jax.experimental import pallas as pl
from jax.experimental.pallas import tpu as pltpu
from jax.experimental.pallas import tpu_sc as plsc

N = 10000
D = 128
E = 320000
NW = 32
CH = 128
KM = 128
K0 = 116
K1 = 42
EPAD = 16 * (K0 + K1) * CH
A = 10112
ZR = A // 16

_mesh = plsc.VectorSubcoreMesh(core_axis_name="c", subcore_axis_name="s")


@functools.partial(
    pl.kernel,
    out_type=jax.ShapeDtypeStruct((2, A, D), jnp.float32),
    mesh=_mesh,
    scratch_types=[
        pltpu.VMEM((KM // 2, CH), jnp.int32),
        pltpu.VMEM((KM // 2, CH), jnp.int32),
        pltpu.VMEM((CH,), jnp.int32),
        pltpu.VMEM((CH,), jnp.int32),
        pltpu.VMEM((CH,), jnp.int32),
        pltpu.VMEM((CH,), jnp.int32),
        pltpu.VMEM((CH, D), jnp.float32),
        pltpu.VMEM((CH, D), jnp.float32),
        pltpu.VMEM_SHARED((A, D), jnp.float32),
        pltpu.SemaphoreType.DMA,
        pltpu.SemaphoreType.DMA,
        pltpu.SemaphoreType.DMA,
        pltpu.SemaphoreType.DMA,
    ],
)
def _sc_segsum(p_hbm, src_hbm, dst_hbm, zeros_hbm, out_hbm,
               src_v, dst_v, sstag0_v, sstag1_v, dstag0_v, dstag1_v,
               rows0_v, rows1_v, acc_sh, semg0, semg1, sems0, sems1):
    c = lax.axis_index("c")
    s = lax.axis_index("s")
    wid = s * 2 + c
    kc2 = jnp.where(c == 0, K0 // 2, K1 // 2)
    pltpu.sync_copy(zeros_hbm, acc_sh.at[pl.ds(s * ZR, ZR)])
    pltpu.sync_copy(src_hbm.at[pl.ds(wid * (KM // 2), KM // 2), :], src_v)
    pltpu.sync_copy(dst_hbm.at[pl.ds(wid * (KM // 2), KM // 2), :], dst_v)
    plsc.subcore_barrier()

    def unpack(ref, i, col0, stag):
        for t in range(4):
            w = ref[i, pl.ds(col0 + t * 16, 16)]
            stag[pl.ds(t * 16, 16)] = lax.bitwise_and(w, 0xFFFF)
            stag[pl.ds(64 + t * 16, 16)] = lax.shift_right_logical(w, 16)

    unpack(src_v, 0, 0, sstag0_v)
    pltpu.async_copy(p_hbm.at[sstag0_v], rows0_v, semg0)
    unpack(src_v, 0, 64, sstag1_v)
    pltpu.async_copy(p_hbm.at[sstag1_v], rows1_v, semg1)

    def body(i, carry):
        ii = lax.rem(i + 1, kc2)
        pltpu.make_async_copy(p_hbm.at[sstag0_v], rows0_v, semg0).wait()
        unpack(dst_v, i, 0, dstag0_v)
        pltpu.async_copy(rows0_v, acc_sh.at[dstag0_v], sems0, add=True)
        unpack(src_v, ii, 0, sstag0_v)
        pltpu.make_async_copy(rows0_v, acc_sh.at[dstag0_v], sems0).wait()
        pltpu.async_copy(p_hbm.at[sstag0_v], rows0_v, semg0)
        pltpu.make_async_copy(p_hbm.at[sstag1_v], rows1_v, semg1).wait()
        unpack(dst_v, i, 64, dstag1_v)
        pltpu.async_copy(rows1_v, acc_sh.at[dstag1_v], sems1, add=True)
        unpack(src_v, ii, 64, sstag1_v)
        pltpu.make_async_copy(rows1_v, acc_sh.at[dstag1_v], sems1).wait()
        pltpu.async_copy(p_hbm.at[sstag1_v], rows1_v, semg1)
        return carry

    lax.fori_loop(0, kc2, body, 0)
    pltpu.make_async_copy(p_hbm.at[sstag0_v], rows0_v, semg0).wait()
    pltpu.make_async_copy(p_hbm.at[sstag1_v], rows1_v, semg1).wait()
    plsc.subcore_barrier()
    pltpu.sync_copy(acc_sh.at[pl.ds(s * ZR, ZR)],
                    out_hbm.at[c, pl.ds(s * ZR, ZR)])


_R = 400


def _tc_stage1_body(x_ref, ax_ref, wra_ref, wrb_ref, woa_ref, wob_ref, b_ref,
                    p_ref, root_ref):
    x = x_ref[...]
    ax = ax_ref[...]
    p_ref[...] = (jnp.dot(x, wra_ref[...], preferred_element_type=jnp.float32)
                  + jnp.dot(ax, wrb_ref[...], preferred_element_type=jnp.float32))
    root_ref[...] = (jnp.dot(x, woa_ref[...], preferred_element_type=jnp.float32)
                     + jnp.dot(ax, wob_ref[...], preferred_element_type=jnp.float32)
                     + b_ref[...])


def _tc_mid_body(a_ref, r_ref, wr_ref, wo_ref, b_ref, p_ref, root_ref):
    h = jnp.maximum(a_ref[0] + a_ref[1] + r_ref[...], 0.0)
    p_ref[...] = jnp.dot(h, wr_ref[...], preferred_element_type=jnp.float32)
    root_ref[...] = (jnp.dot(h, wo_ref[...], preferred_element_type=jnp.float32)
                     + b_ref[...])


def _tc_final_body(a_ref, r_ref, ax_ref, o_ref):
    o_ref[...] = a_ref[0] + a_ref[1] + r_ref[...] + ax_ref[...]


def _rows_spec(din):
    return pl.BlockSpec((_R, din), lambda i: (i, 0))


def _full_spec(din):
    return pl.BlockSpec((din, D), lambda i: (0, 0))


_B_SPEC = pl.BlockSpec((1, D), lambda i: (0, 0))
_ACC_SPEC = pl.BlockSpec((2, _R, D), lambda i: (0, i, 0))


def _tc_stage1(x, ax, wr, wo, b):
    return pl.pallas_call(
        _tc_stage1_body,
        grid=(N // _R,),
        in_specs=[_rows_spec(D), _rows_spec(D),
                  _full_spec(D), _full_spec(D),
                  _full_spec(D), _full_spec(D), _B_SPEC],
        out_specs=[_rows_spec(D), _rows_spec(D)],
        out_shape=[jax.ShapeDtypeStruct((N, D), jnp.float32),
                   jax.ShapeDtypeStruct((N, D), jnp.float32)],
    )(x, ax, wr[:D], wr[D:], wo[:D], wo[D:], b)


def _tc_mid(acc, r, wr, wo, b):
    return pl.pallas_call(
        _tc_mid_body,
        grid=(N // _R,),
        in_specs=[_ACC_SPEC, _rows_spec(D),
                  _full_spec(D), _full_spec(D), _B_SPEC],
        out_specs=[_rows_spec(D), _rows_spec(D)],
        out_shape=[jax.ShapeDtypeStruct((N, D), jnp.float32),
                   jax.ShapeDtypeStruct((N, D), jnp.float32)],
    )(acc, r, wr, wo, b)


def _tc_final(acc, r, ax):
    return pl.pallas_call(
        _tc_final_body,
        grid=(N // _R,),
        in_specs=[_ACC_SPEC, _rows_spec(D), _rows_spec(D)],
        out_specs=_rows_spec(D),
        out_shape=jax.ShapeDtypeStruct((N, D), jnp.float32),
    )(acc, r, ax)


def kernel(x, edge_index, additional_x, W_rel1, W_root1, b1,
           W_rel2, W_root2, b2, W_rel3, W_root3, b3):
    src = edge_index[0]
    dst = edge_index[1]
    pad = EPAD - E

    def edge_layout(idx, fill):
        flat = jnp.concatenate([idx, jnp.full((pad,), fill, jnp.int32)])
        cut = 16 * K0 * CH
        e0 = flat[:cut].reshape(16, K0, CH)
        e1 = flat[cut:].reshape(16, K1, CH)
        e0 = jnp.pad(e0, ((0, 0), (0, KM - K0), (0, 0)), constant_values=fill)
        e1 = jnp.pad(e1, ((0, 0), (0, KM - K1), (0, 0)), constant_values=fill)
        e = jnp.stack([e0, e1], axis=1).reshape(NW, KM, CH)
        return (e[:, :, :64] | (e[:, :, 64:] << 16)).reshape(NW * (KM // 2), CH)

    src_p = edge_layout(src, 0)
    dst_p = edge_layout(dst, N)
    zeros = jnp.zeros((ZR, D), jnp.float32)

    p1, root1 = _tc_stage1(x, additional_x, W_rel1, W_root1, b1.reshape(1, D))
    acc = _sc_segsum(p1, src_p, dst_p, zeros)
    p2, root2 = _tc_mid(acc, root1, W_rel2, W_root2, b2.reshape(1, D))
    acc = _sc_segsum(p2, src_p, dst_p, zeros)
    p3, root3 = _tc_mid(acc, root2, W_rel3, W_root3, b3.reshape(1, D))
    acc = _sc_segsum(p3, src_p, dst_p, zeros)
    return _tc_final(acc, root3, additional_x)
